# Initial kernel scaffold; baseline (speedup 1.0000x reference)
import jax
import jax.numpy as jnp
from jax.experimental import pallas as pl


def kernel(higher_feats, lower_points, higher_points):
    raise NotImplementedError("write your pallas kernel here")


# speedup vs baseline: 15.4213x; 15.4213x over previous
import functools
import jax
import jax.numpy as jnp
from jax import lax
from jax.experimental import pallas as pl
from jax.experimental.pallas import tpu as pltpu
from jax.experimental.pallas import tpu_sc as plsc


def _topk_weights_kernel(lp_ref, hpt_ref, idx_ref, w0_ref, w1_ref, w2_ref):
    b = pl.program_id(0)
    lp = lp_ref[0]
    hpt = hpt_ref[0]
    bq = lp.shape[0]
    m = hpt.shape[1]

    lpb = lp.astype(jnp.bfloat16).astype(jnp.float32)
    hptb = hpt.astype(jnp.bfloat16).astype(jnp.float32)
    s = lpb[:, 0:1] * hptb[0:1, :]
    s = s + lpb[:, 1:2] * hptb[1:2, :]
    s = s + lpb[:, 2:3] * hptb[2:3, :]
    d = -2.0 * s
    lpsq = lp[:, 0:1] * lp[:, 0:1]
    lpsq = lpsq + lp[:, 1:2] * lp[:, 1:2]
    lpsq = lpsq + lp[:, 2:3] * lp[:, 2:3]
    hpsq = hpt[0:1, :] * hpt[0:1, :]
    hpsq = hpsq + hpt[1:2, :] * hpt[1:2, :]
    hpsq = hpsq + hpt[2:3, :] * hpt[2:3, :]
    d = d + lpsq
    d = d + hpsq

    iota = lax.broadcasted_iota(jnp.int32, (bq, m), 1)
    inf = jnp.float32(jnp.inf)
    cur = d
    vals, idxs = [], []
    for _ in range(3):
        mv = jnp.min(cur, axis=1, keepdims=True)
        mi = jnp.min(jnp.where(cur == mv, iota, m), axis=1, keepdims=True)
        vals.append(mv)
        idxs.append(mi)
        cur = jnp.where(iota == mi, inf, cur)

    recip = [1.0 / (v + 1e-8) for v in vals]
    norm = recip[0] + recip[1] + recip[2]
    w = [r / norm for r in recip]

    idx_ref[0] = jnp.concatenate(idxs, axis=1) + b * m
    w0_ref[0] = jnp.broadcast_to(w[0], (bq, 16))
    w1_ref[0] = jnp.broadcast_to(w[1], (bq, 16))
    w2_ref[0] = jnp.broadcast_to(w[2], (bq, 16))


def _tc_stage(lower_points, higher_points, M):
    B, N, _ = lower_points.shape
    BQ = 512
    hpt = jnp.swapaxes(higher_points, 1, 2)
    wspec = pl.BlockSpec((1, BQ, 16), lambda b, i: (b, i, 0))
    wshape = jax.ShapeDtypeStruct((B, N, 16), jnp.float32)
    return pl.pallas_call(
        _topk_weights_kernel,
        grid=(B, N // BQ),
        in_specs=[
            pl.BlockSpec((1, BQ, 3), lambda b, i: (b, i, 0)),
            pl.BlockSpec((1, 3, M), lambda b, i: (b, 0, 0)),
        ],
        out_specs=[
            pl.BlockSpec((1, BQ, 3), lambda b, i: (b, i, 0)),
            wspec, wspec, wspec,
        ],
        out_shape=[
            jax.ShapeDtypeStruct((B, N, 3), jnp.int32),
            wshape, wshape, wshape,
        ],
    )(lower_points, hpt)


_NC, _NS, _LANES = 2, 16, 16
_NW = _NC * _NS


def _make_sc_gather(Q, C, CQ):
    qpw = Q // _NW
    nchunk = qpw // CQ
    mesh = plsc.VectorSubcoreMesh(core_axis_name="c", subcore_axis_name="s")

    @functools.partial(
        pl.kernel,
        out_type=jax.ShapeDtypeStruct((Q, C), jnp.float32),
        mesh=mesh,
        scratch_types=[
            pltpu.VMEM((3 * CQ,), jnp.int32),
            pltpu.VMEM((CQ, 16), jnp.float32),
            pltpu.VMEM((CQ, 16), jnp.float32),
            pltpu.VMEM((CQ, 16), jnp.float32),
            pltpu.VMEM((3 * CQ, C), jnp.float32),
            pltpu.VMEM((CQ, C), jnp.float32),
            pltpu.SemaphoreType.DMA,
        ],
    )
    def sc_gather(feats_hbm, idx_hbm, w0_hbm, w1_hbm, w2_hbm, out_hbm,
                  idx_v, w0_v, w1_v, w2_v, rows_v, out_v, sem):
        wid = lax.axis_index("s") * _NC + lax.axis_index("c")
        base_q = wid * qpw

        def chunk_body(ci, carry):
            q0 = base_q + ci * CQ
            pltpu.sync_copy(idx_hbm.at[pl.ds(q0 * 3, 3 * CQ)], idx_v)
            pltpu.sync_copy(w0_hbm.at[pl.ds(q0, CQ)], w0_v)
            pltpu.sync_copy(w1_hbm.at[pl.ds(q0, CQ)], w1_v)
            pltpu.sync_copy(w2_hbm.at[pl.ds(q0, CQ)], w2_v)
            pltpu.async_copy(feats_hbm.at[idx_v], rows_v, sem).wait()
            for q in range(CQ):
                w0 = w0_v[q, :]
                w1 = w1_v[q, :]
                w2 = w2_v[q, :]
                for dc in range(C // _LANES):
                    sl = pl.ds(dc * _LANES, _LANES)
                    acc = rows_v[3 * q, sl] * w0
                    acc = acc + rows_v[3 * q + 1, sl] * w1
                    acc = acc + rows_v[3 * q + 2, sl] * w2
                    out_v[q, sl] = acc
            pltpu.sync_copy(out_v, out_hbm.at[pl.ds(q0, CQ)])
            return carry

        lax.fori_loop(0, nchunk, chunk_body, 0)

    return sc_gather


def kernel(higher_feats, lower_points, higher_points):
    B, N, _ = lower_points.shape
    _, M, C = higher_feats.shape
    Q = B * N
    idx, w0, w1, w2 = _tc_stage(lower_points, higher_points, M)
    feats_flat = higher_feats.reshape(B * M, C)
    idx_flat = idx.reshape(Q * 3)
    out = _make_sc_gather(Q, C, 32)(
        feats_flat, idx_flat,
        w0.reshape(Q, 16), w1.reshape(Q, 16), w2.reshape(Q, 16))
    return out.reshape(B, N, C)


# speedup vs baseline: 22.8810x; 1.4837x over previous
import functools
import jax
import jax.numpy as jnp
from jax import lax
from jax.experimental import pallas as pl
from jax.experimental.pallas import tpu as pltpu
from jax.experimental.pallas import tpu_sc as plsc


def _topk_weights_kernel(lp_ref, hpt_ref, idx_ref, w_ref):
    b = pl.program_id(0)
    lp = lp_ref[0]
    hpt = hpt_ref[0]
    bq = lp.shape[0]
    m = hpt.shape[1]

    lpb = lp.astype(jnp.bfloat16).astype(jnp.float32)
    hptb = hpt.astype(jnp.bfloat16).astype(jnp.float32)
    s = lpb[:, 0:1] * hptb[0:1, :]
    s = s + lpb[:, 1:2] * hptb[1:2, :]
    s = s + lpb[:, 2:3] * hptb[2:3, :]
    d = -2.0 * s
    lpsq = lp[:, 0:1] * lp[:, 0:1]
    lpsq = lpsq + lp[:, 1:2] * lp[:, 1:2]
    lpsq = lpsq + lp[:, 2:3] * lp[:, 2:3]
    hpsq = hpt[0:1, :] * hpt[0:1, :]
    hpsq = hpsq + hpt[1:2, :] * hpt[1:2, :]
    hpsq = hpsq + hpt[2:3, :] * hpt[2:3, :]
    d = d + lpsq
    d = d + hpsq

    iota = lax.broadcasted_iota(jnp.int32, (bq, m), 1)
    inf = jnp.float32(jnp.inf)
    cur = d
    vals, idxs = [], []
    for _ in range(3):
        mv = jnp.min(cur, axis=1, keepdims=True)
        mi = jnp.min(jnp.where(cur == mv, iota, m), axis=1, keepdims=True)
        vals.append(mv)
        idxs.append(mi)
        cur = jnp.where(iota == mi, inf, cur)

    recip = [1.0 / (v + 1e-8) for v in vals]
    norm = recip[0] + recip[1] + recip[2]
    w = [r / norm for r in recip]

    idx_ref[0] = jnp.concatenate(idxs, axis=1) + b * m
    w_ref[0] = jnp.concatenate(
        [jnp.broadcast_to(wk, (bq, 16)) for wk in w], axis=1)


def _tc_stage(lower_points, higher_points, M):
    B, N, _ = lower_points.shape
    BQ = 512
    hpt = jnp.swapaxes(higher_points, 1, 2)
    return pl.pallas_call(
        _topk_weights_kernel,
        grid=(B, N // BQ),
        in_specs=[
            pl.BlockSpec((1, BQ, 3), lambda b, i: (b, i, 0)),
            pl.BlockSpec((1, 3, M), lambda b, i: (b, 0, 0)),
        ],
        out_specs=[
            pl.BlockSpec((1, BQ, 3), lambda b, i: (b, i, 0)),
            pl.BlockSpec((1, BQ, 48), lambda b, i: (b, i, 0)),
        ],
        out_shape=[
            jax.ShapeDtypeStruct((B, N, 3), jnp.int32),
            jax.ShapeDtypeStruct((B, N, 48), jnp.float32),
        ],
    )(lower_points, hpt)


_NC, _NS, _LANES = 2, 16, 16
_NW = _NC * _NS


def _make_sc_gather(Q, C, CQ):
    qpw = Q // _NW
    nchunk = qpw // CQ
    assert nchunk % 2 == 0
    mesh = plsc.VectorSubcoreMesh(core_axis_name="c", subcore_axis_name="s")

    @functools.partial(
        pl.kernel,
        out_type=jax.ShapeDtypeStruct((Q, C), jnp.float32),
        mesh=mesh,
        scratch_types=[
            pltpu.VMEM((3 * qpw,), jnp.int32),
            pltpu.VMEM((CQ, 48), jnp.float32),
            pltpu.VMEM((CQ, 48), jnp.float32),
            pltpu.VMEM((3 * CQ, C), jnp.float32),
            pltpu.VMEM((3 * CQ, C), jnp.float32),
            pltpu.VMEM((CQ, C), jnp.float32),
            pltpu.SemaphoreType.DMA,
            pltpu.SemaphoreType.DMA,
            pltpu.SemaphoreType.DMA,
            pltpu.SemaphoreType.DMA,
        ],
    )
    def sc_gather(feats_hbm, idx_hbm, w_hbm, out_hbm,
                  idx_v, w_va, w_vb, rows_a, rows_b, out_v,
                  gsem_a, gsem_b, wsem_a, wsem_b):
        wid = lax.axis_index("s") * _NC + lax.axis_index("c")
        base_q = wid * qpw

        pltpu.sync_copy(idx_hbm.at[pl.ds(base_q * 3, 3 * qpw)], idx_v)

        def chunk_start(cc, rows, w_v, gsem, wsem):
            pltpu.async_copy(
                feats_hbm.at[idx_v.at[pl.ds(cc * 3 * CQ, 3 * CQ)]],
                rows, gsem)
            pltpu.async_copy(
                w_hbm.at[pl.ds(base_q + cc * CQ, CQ)], w_v, wsem)

        def chunk_wait(cc, rows, w_v, gsem, wsem):
            pltpu.make_async_copy(
                feats_hbm.at[idx_v.at[pl.ds(cc * 3 * CQ, 3 * CQ)]],
                rows, gsem).wait()
            pltpu.make_async_copy(
                w_hbm.at[pl.ds(base_q + cc * CQ, CQ)], w_v, wsem).wait()

        def compute_store(cc, rows, w_v):
            for q in range(CQ):
                w0 = w_v[q, pl.ds(0, 16)]
                w1 = w_v[q, pl.ds(16, 16)]
                w2 = w_v[q, pl.ds(32, 16)]
                for dc in range(C // _LANES):
                    sl = pl.ds(dc * _LANES, _LANES)
                    acc = rows[3 * q, sl] * w0
                    acc = acc + rows[3 * q + 1, sl] * w1
                    acc = acc + rows[3 * q + 2, sl] * w2
                    out_v[q, sl] = acc
            pltpu.sync_copy(out_v, out_hbm.at[pl.ds(base_q + cc * CQ, CQ)])

        chunk_start(0, rows_a, w_va, gsem_a, wsem_a)
        chunk_start(1, rows_b, w_vb, gsem_b, wsem_b)

        def pair_body(i, carry):
            cc = 2 * i
            chunk_wait(cc, rows_a, w_va, gsem_a, wsem_a)
            compute_store(cc, rows_a, w_va)

            @pl.when(cc + 2 < nchunk)
            def _():
                chunk_start(cc + 2, rows_a, w_va, gsem_a, wsem_a)

            chunk_wait(cc + 1, rows_b, w_vb, gsem_b, wsem_b)
            compute_store(cc + 1, rows_b, w_vb)

            @pl.when(cc + 3 < nchunk)
            def _():
                chunk_start(cc + 3, rows_b, w_vb, gsem_b, wsem_b)

            return carry

        lax.fori_loop(0, nchunk // 2, pair_body, 0)

    return sc_gather


def kernel(higher_feats, lower_points, higher_points):
    B, N, _ = lower_points.shape
    _, M, C = higher_feats.shape
    Q = B * N
    idx, w = _tc_stage(lower_points, higher_points, M)
    feats_flat = higher_feats.reshape(B * M, C)
    idx_flat = idx.reshape(Q * 3)
    out = _make_sc_gather(Q, C, 16)(feats_flat, idx_flat, w.reshape(Q, 48))
    return out.reshape(B, N, C)


# speedup vs baseline: 23.3159x; 1.0190x over previous
import functools
import jax
import jax.numpy as jnp
from jax import lax
from jax.experimental import pallas as pl
from jax.experimental.pallas import tpu as pltpu
from jax.experimental.pallas import tpu_sc as plsc


def _topk_weights_kernel(lp_ref, hpt_ref, idx_ref, w_ref):
    b = pl.program_id(0)
    lp = lp_ref[0]
    hpt = hpt_ref[0]
    bq = lp.shape[0]
    m = hpt.shape[1]

    lpb = lp.astype(jnp.bfloat16).astype(jnp.float32)
    hptb = hpt.astype(jnp.bfloat16).astype(jnp.float32)
    s = lpb[:, 0:1] * hptb[0:1, :]
    s = s + lpb[:, 1:2] * hptb[1:2, :]
    s = s + lpb[:, 2:3] * hptb[2:3, :]
    d = -2.0 * s
    lpsq = lp[:, 0:1] * lp[:, 0:1]
    lpsq = lpsq + lp[:, 1:2] * lp[:, 1:2]
    lpsq = lpsq + lp[:, 2:3] * lp[:, 2:3]
    hpsq = hpt[0:1, :] * hpt[0:1, :]
    hpsq = hpsq + hpt[1:2, :] * hpt[1:2, :]
    hpsq = hpsq + hpt[2:3, :] * hpt[2:3, :]
    d = d + lpsq
    d = d + hpsq

    iota = lax.broadcasted_iota(jnp.int32, (bq, m), 1)
    inf = jnp.float32(jnp.inf)
    cur = d
    vals, idxs = [], []
    for _ in range(3):
        mv = jnp.min(cur, axis=1, keepdims=True)
        mi = jnp.min(jnp.where(cur == mv, iota, m), axis=1, keepdims=True)
        vals.append(mv)
        idxs.append(mi)
        cur = jnp.where(iota == mi, inf, cur)

    recip = [1.0 / (v + 1e-8) for v in vals]
    norm = recip[0] + recip[1] + recip[2]
    w = [r / norm for r in recip]

    idx_ref[0] = jnp.concatenate(idxs, axis=1) + b * m
    w_ref[0] = jnp.concatenate(
        [jnp.broadcast_to(wk, (bq, 16)) for wk in w], axis=1)


def _tc_stage(lower_points, higher_points, M):
    B, N, _ = lower_points.shape
    BQ = 512
    hpt = jnp.swapaxes(higher_points, 1, 2)
    return pl.pallas_call(
        _topk_weights_kernel,
        grid=(B, N // BQ),
        in_specs=[
            pl.BlockSpec((1, BQ, 3), lambda b, i: (b, i, 0)),
            pl.BlockSpec((1, 3, M), lambda b, i: (b, 0, 0)),
        ],
        out_specs=[
            pl.BlockSpec((1, BQ, 3), lambda b, i: (b, i, 0)),
            pl.BlockSpec((1, BQ, 48), lambda b, i: (b, i, 0)),
        ],
        out_shape=[
            jax.ShapeDtypeStruct((B, N, 3), jnp.int32),
            jax.ShapeDtypeStruct((B, N, 48), jnp.float32),
        ],
    )(lower_points, hpt)


_NC, _NS, _LANES = 2, 16, 16
_NW = _NC * _NS


def _make_sc_gather(Q, C, CQ):
    qpw = Q // _NW
    nchunk = qpw // CQ
    assert nchunk % 2 == 0
    mesh = plsc.VectorSubcoreMesh(core_axis_name="c", subcore_axis_name="s")

    @functools.partial(
        pl.kernel,
        out_type=jax.ShapeDtypeStruct((Q, C), jnp.float32),
        mesh=mesh,
        scratch_types=[
            pltpu.VMEM((3 * qpw,), jnp.int32),
            pltpu.VMEM((CQ, 48), jnp.float32),
            pltpu.VMEM((CQ, 48), jnp.float32),
            pltpu.VMEM((3 * CQ, C), jnp.float32),
            pltpu.VMEM((3 * CQ, C), jnp.float32),
            pltpu.VMEM((CQ, C), jnp.float32),
            pltpu.VMEM((CQ, C), jnp.float32),
            pltpu.SemaphoreType.DMA,
            pltpu.SemaphoreType.DMA,
            pltpu.SemaphoreType.DMA,
            pltpu.SemaphoreType.DMA,
            pltpu.SemaphoreType.DMA,
            pltpu.SemaphoreType.DMA,
        ],
    )
    def sc_gather(feats_hbm, idx_hbm, w_hbm, out_hbm,
                  idx_v, w_va, w_vb, rows_a, rows_b, out_va, out_vb,
                  gsem_a, gsem_b, wsem_a, wsem_b, osem_a, osem_b):
        wid = lax.axis_index("s") * _NC + lax.axis_index("c")
        base_q = wid * qpw

        pltpu.sync_copy(idx_hbm.at[pl.ds(base_q * 3, 3 * qpw)], idx_v)

        def chunk_start(cc, rows, w_v, gsem, wsem):
            pltpu.async_copy(
                feats_hbm.at[idx_v.at[pl.ds(cc * 3 * CQ, 3 * CQ)]],
                rows, gsem)
            pltpu.async_copy(
                w_hbm.at[pl.ds(base_q + cc * CQ, CQ)], w_v, wsem)

        def chunk_wait(cc, rows, w_v, gsem, wsem):
            pltpu.make_async_copy(
                feats_hbm.at[idx_v.at[pl.ds(cc * 3 * CQ, 3 * CQ)]],
                rows, gsem).wait()
            pltpu.make_async_copy(
                w_hbm.at[pl.ds(base_q + cc * CQ, CQ)], w_v, wsem).wait()

        def out_wait(cc, out_v, osem):
            pltpu.make_async_copy(
                out_v, out_hbm.at[pl.ds(base_q + cc * CQ, CQ)], osem).wait()

        def compute_store(cc, rows, w_v, out_v, osem):
            @pl.when(cc >= 2)
            def _():
                out_wait(cc - 2, out_v, osem)

            for q in range(CQ):
                w0 = w_v[q, pl.ds(0, 16)]
                w1 = w_v[q, pl.ds(16, 16)]
                w2 = w_v[q, pl.ds(32, 16)]
                for dc in range(C // _LANES):
                    sl = pl.ds(dc * _LANES, _LANES)
                    acc = rows[3 * q, sl] * w0
                    acc = acc + rows[3 * q + 1, sl] * w1
                    acc = acc + rows[3 * q + 2, sl] * w2
                    out_v[q, sl] = acc
            pltpu.async_copy(
                out_v, out_hbm.at[pl.ds(base_q + cc * CQ, CQ)], osem)

        chunk_start(0, rows_a, w_va, gsem_a, wsem_a)
        chunk_start(1, rows_b, w_vb, gsem_b, wsem_b)

        def pair_body(i, carry):
            cc = 2 * i
            chunk_wait(cc, rows_a, w_va, gsem_a, wsem_a)
            compute_store(cc, rows_a, w_va, out_va, osem_a)

            @pl.when(cc + 2 < nchunk)
            def _():
                chunk_start(cc + 2, rows_a, w_va, gsem_a, wsem_a)

            chunk_wait(cc + 1, rows_b, w_vb, gsem_b, wsem_b)
            compute_store(cc + 1, rows_b, w_vb, out_vb, osem_b)

            @pl.when(cc + 3 < nchunk)
            def _():
                chunk_start(cc + 3, rows_b, w_vb, gsem_b, wsem_b)

            return carry

        lax.fori_loop(0, nchunk // 2, pair_body, 0)
        out_wait(nchunk - 2, out_va, osem_a)
        out_wait(nchunk - 1, out_vb, osem_b)

    return sc_gather


def kernel(higher_feats, lower_points, higher_points):
    B, N, _ = lower_points.shape
    _, M, C = higher_feats.shape
    Q = B * N
    idx, w = _tc_stage(lower_points, higher_points, M)
    feats_flat = higher_feats.reshape(B * M, C)
    idx_flat = idx.reshape(Q * 3)
    out = _make_sc_gather(Q, C, 16)(feats_flat, idx_flat, w.reshape(Q, 48))
    return out.reshape(B, N, C)


# speedup vs baseline: 23.4390x; 1.0053x over previous
import functools
import jax
import jax.numpy as jnp
from jax import lax
from jax.experimental import pallas as pl
from jax.experimental.pallas import tpu as pltpu
from jax.experimental.pallas import tpu_sc as plsc


def _topk_weights_kernel(lp_ref, hpt_ref, idx_ref, w_ref):
    b = pl.program_id(0)
    lp = lp_ref[0]
    hpt = hpt_ref[0]
    bq = lp.shape[0]
    m = hpt.shape[1]

    lpb = lp.astype(jnp.bfloat16).astype(jnp.float32)
    hptb = hpt.astype(jnp.bfloat16).astype(jnp.float32)
    s = lpb[:, 0:1] * hptb[0:1, :]
    s = s + lpb[:, 1:2] * hptb[1:2, :]
    s = s + lpb[:, 2:3] * hptb[2:3, :]
    d = -2.0 * s
    lpsq = lp[:, 0:1] * lp[:, 0:1]
    lpsq = lpsq + lp[:, 1:2] * lp[:, 1:2]
    lpsq = lpsq + lp[:, 2:3] * lp[:, 2:3]
    hpsq = hpt[0:1, :] * hpt[0:1, :]
    hpsq = hpsq + hpt[1:2, :] * hpt[1:2, :]
    hpsq = hpsq + hpt[2:3, :] * hpt[2:3, :]
    d = d + lpsq
    d = d + hpsq

    iota = lax.broadcasted_iota(jnp.int32, (bq, m), 1)
    inf = jnp.float32(jnp.inf)
    cur = d
    vals, idxs = [], []
    for _ in range(3):
        mv = jnp.min(cur, axis=1, keepdims=True)
        mi = jnp.min(jnp.where(cur == mv, iota, m), axis=1, keepdims=True)
        vals.append(mv)
        idxs.append(mi)
        cur = jnp.where(iota == mi, inf, cur)

    recip = [1.0 / (v + 1e-8) for v in vals]
    norm = recip[0] + recip[1] + recip[2]
    w = [r / norm for r in recip]

    idx_ref[0] = jnp.concatenate(idxs, axis=1) + b * m
    w_ref[0] = jnp.concatenate(
        [jnp.broadcast_to(wk, (bq, 16)) for wk in w], axis=1)


def _tc_stage(lower_points, higher_points, M):
    B, N, _ = lower_points.shape
    BQ = 1024
    hpt = jnp.swapaxes(higher_points, 1, 2)
    return pl.pallas_call(
        _topk_weights_kernel,
        grid=(B, N // BQ),
        in_specs=[
            pl.BlockSpec((1, BQ, 3), lambda b, i: (b, i, 0)),
            pl.BlockSpec((1, 3, M), lambda b, i: (b, 0, 0)),
        ],
        out_specs=[
            pl.BlockSpec((1, BQ, 3), lambda b, i: (b, i, 0)),
            pl.BlockSpec((1, BQ, 48), lambda b, i: (b, i, 0)),
        ],
        out_shape=[
            jax.ShapeDtypeStruct((B, N, 3), jnp.int32),
            jax.ShapeDtypeStruct((B, N, 48), jnp.float32),
        ],
    )(lower_points, hpt)


_NC, _NS, _LANES = 2, 16, 16
_NW = _NC * _NS


def _make_sc_gather(Q, C, CQ):
    qpw = Q // _NW
    nchunk = qpw // CQ
    assert nchunk % 2 == 0
    mesh = plsc.VectorSubcoreMesh(core_axis_name="c", subcore_axis_name="s")

    @functools.partial(
        pl.kernel,
        out_type=jax.ShapeDtypeStruct((Q, C), jnp.float32),
        mesh=mesh,
        scratch_types=[
            pltpu.VMEM((3 * qpw,), jnp.int32),
            pltpu.VMEM((CQ, 48), jnp.float32),
            pltpu.VMEM((CQ, 48), jnp.float32),
            pltpu.VMEM((3 * CQ, C), jnp.float32),
            pltpu.VMEM((3 * CQ, C), jnp.float32),
            pltpu.VMEM((CQ, C), jnp.float32),
            pltpu.VMEM((CQ, C), jnp.float32),
            pltpu.SemaphoreType.DMA,
            pltpu.SemaphoreType.DMA,
            pltpu.SemaphoreType.DMA,
            pltpu.SemaphoreType.DMA,
            pltpu.SemaphoreType.DMA,
            pltpu.SemaphoreType.DMA,
        ],
    )
    def sc_gather(feats_hbm, idx_hbm, w_hbm, out_hbm,
                  idx_v, w_va, w_vb, rows_a, rows_b, out_va, out_vb,
                  gsem_a, gsem_b, wsem_a, wsem_b, osem_a, osem_b):
        wid = lax.axis_index("s") * _NC + lax.axis_index("c")
        base_q = wid * qpw

        pltpu.sync_copy(idx_hbm.at[pl.ds(base_q * 3, 3 * qpw)], idx_v)

        def chunk_start(cc, rows, w_v, gsem, wsem):
            pltpu.async_copy(
                feats_hbm.at[idx_v.at[pl.ds(cc * 3 * CQ, 3 * CQ)]],
                rows, gsem)
            pltpu.async_copy(
                w_hbm.at[pl.ds(base_q + cc * CQ, CQ)], w_v, wsem)

        def chunk_wait(cc, rows, w_v, gsem, wsem):
            pltpu.make_async_copy(
                feats_hbm.at[idx_v.at[pl.ds(cc * 3 * CQ, 3 * CQ)]],
                rows, gsem).wait()
            pltpu.make_async_copy(
                w_hbm.at[pl.ds(base_q + cc * CQ, CQ)], w_v, wsem).wait()

        def out_wait(cc, out_v, osem):
            pltpu.make_async_copy(
                out_v, out_hbm.at[pl.ds(base_q + cc * CQ, CQ)], osem).wait()

        def compute_store(cc, rows, w_v, out_v, osem):
            @pl.when(cc >= 2)
            def _():
                out_wait(cc - 2, out_v, osem)

            for q in range(CQ):
                w0 = w_v[q, pl.ds(0, 16)]
                w1 = w_v[q, pl.ds(16, 16)]
                w2 = w_v[q, pl.ds(32, 16)]
                for dc in range(C // _LANES):
                    sl = pl.ds(dc * _LANES, _LANES)
                    acc = rows[3 * q, sl] * w0
                    acc = acc + rows[3 * q + 1, sl] * w1
                    acc = acc + rows[3 * q + 2, sl] * w2
                    out_v[q, sl] = acc
            pltpu.async_copy(
                out_v, out_hbm.at[pl.ds(base_q + cc * CQ, CQ)], osem)

        chunk_start(0, rows_a, w_va, gsem_a, wsem_a)
        chunk_start(1, rows_b, w_vb, gsem_b, wsem_b)

        def pair_body(i, carry):
            cc = 2 * i
            chunk_wait(cc, rows_a, w_va, gsem_a, wsem_a)
            compute_store(cc, rows_a, w_va, out_va, osem_a)

            @pl.when(cc + 2 < nchunk)
            def _():
                chunk_start(cc + 2, rows_a, w_va, gsem_a, wsem_a)

            chunk_wait(cc + 1, rows_b, w_vb, gsem_b, wsem_b)
            compute_store(cc + 1, rows_b, w_vb, out_vb, osem_b)

            @pl.when(cc + 3 < nchunk)
            def _():
                chunk_start(cc + 3, rows_b, w_vb, gsem_b, wsem_b)

            return carry

        lax.fori_loop(0, nchunk // 2, pair_body, 0)
        out_wait(nchunk - 2, out_va, osem_a)
        out_wait(nchunk - 1, out_vb, osem_b)

    return sc_gather


def kernel(higher_feats, lower_points, higher_points):
    B, N, _ = lower_points.shape
    _, M, C = higher_feats.shape
    Q = B * N
    idx, w = _tc_stage(lower_points, higher_points, M)
    feats_flat = higher_feats.reshape(B * M, C)
    idx_flat = idx.reshape(Q * 3)
    out = _make_sc_gather(Q, C, 16)(feats_flat, idx_flat, w.reshape(Q, 48))
    return out.reshape(B, N, C)


# speedup vs baseline: 28.2900x; 1.2070x over previous
import functools
import jax
import jax.numpy as jnp
from jax import lax
from jax.experimental import pallas as pl
from jax.experimental.pallas import tpu as pltpu
from jax.experimental.pallas import tpu_sc as plsc


def _topk_weights_kernel(lp_ref, hpt_ref, idx_ref, w_ref, *, row_offset):
    lp = lp_ref[...]
    hpt = hpt_ref[...]
    bq = lp.shape[0]
    m = hpt.shape[1]

    lpb = lp.astype(jnp.bfloat16).astype(jnp.float32)
    hptb = hpt.astype(jnp.bfloat16).astype(jnp.float32)
    s = lpb[:, 0:1] * hptb[0:1, :]
    s = s + lpb[:, 1:2] * hptb[1:2, :]
    s = s + lpb[:, 2:3] * hptb[2:3, :]
    d = -2.0 * s
    lpsq = lp[:, 0:1] * lp[:, 0:1]
    lpsq = lpsq + lp[:, 1:2] * lp[:, 1:2]
    lpsq = lpsq + lp[:, 2:3] * lp[:, 2:3]
    hpsq = hpt[0:1, :] * hpt[0:1, :]
    hpsq = hpsq + hpt[1:2, :] * hpt[1:2, :]
    hpsq = hpsq + hpt[2:3, :] * hpt[2:3, :]
    d = d + lpsq
    d = d + hpsq

    iota = lax.broadcasted_iota(jnp.int32, (bq, m), 1)
    inf = jnp.float32(jnp.inf)
    cur = d
    vals, idxs = [], []
    for _ in range(3):
        mv = jnp.min(cur, axis=1, keepdims=True)
        mi = jnp.min(jnp.where(cur == mv, iota, m), axis=1, keepdims=True)
        vals.append(mv)
        idxs.append(mi)
        cur = jnp.where(iota == mi, inf, cur)

    recip = [1.0 / (v + 1e-8) for v in vals]
    norm = recip[0] + recip[1] + recip[2]
    w = [r / norm for r in recip]

    idx_ref[...] = jnp.concatenate(idxs, axis=1) + row_offset
    w_ref[...] = jnp.concatenate(
        [jnp.broadcast_to(wk, (bq, 16)) for wk in w], axis=1)


def _tc_stage_batch(lp_b, hpt_b, M, row_offset):
    N, _ = lp_b.shape
    BQ = 1024
    return pl.pallas_call(
        functools.partial(_topk_weights_kernel, row_offset=row_offset),
        grid=(N // BQ,),
        in_specs=[
            pl.BlockSpec((BQ, 3), lambda i: (i, 0)),
            pl.BlockSpec((3, M), lambda i: (0, 0)),
        ],
        out_specs=[
            pl.BlockSpec((BQ, 3), lambda i: (i, 0)),
            pl.BlockSpec((BQ, 48), lambda i: (i, 0)),
        ],
        out_shape=[
            jax.ShapeDtypeStruct((N, 3), jnp.int32),
            jax.ShapeDtypeStruct((N, 48), jnp.float32),
        ],
    )(lp_b, hpt_b)


_NC, _NS, _LANES = 2, 16, 16
_NW = _NC * _NS


def _make_sc_gather(Q, C, CQ):
    qpw = Q // _NW
    nchunk = qpw // CQ
    assert nchunk % 2 == 0
    mesh = plsc.VectorSubcoreMesh(core_axis_name="c", subcore_axis_name="s")

    @functools.partial(
        pl.kernel,
        out_type=jax.ShapeDtypeStruct((Q, C), jnp.float32),
        mesh=mesh,
        scratch_types=[
            pltpu.VMEM((3 * qpw,), jnp.int32),
            pltpu.VMEM((CQ, 48), jnp.float32),
            pltpu.VMEM((CQ, 48), jnp.float32),
            pltpu.VMEM((3 * CQ, C), jnp.float32),
            pltpu.VMEM((3 * CQ, C), jnp.float32),
            pltpu.VMEM((CQ, C), jnp.float32),
            pltpu.VMEM((CQ, C), jnp.float32),
            pltpu.SemaphoreType.DMA,
            pltpu.SemaphoreType.DMA,
            pltpu.SemaphoreType.DMA,
            pltpu.SemaphoreType.DMA,
            pltpu.SemaphoreType.DMA,
            pltpu.SemaphoreType.DMA,
        ],
    )
    def sc_gather(feats_hbm, idx_hbm, w_hbm, out_hbm,
                  idx_v, w_va, w_vb, rows_a, rows_b, out_va, out_vb,
                  gsem_a, gsem_b, wsem_a, wsem_b, osem_a, osem_b):
        wid = lax.axis_index("s") * _NC + lax.axis_index("c")
        base_q = wid * qpw

        pltpu.sync_copy(idx_hbm.at[pl.ds(base_q * 3, 3 * qpw)], idx_v)

        def chunk_start(cc, rows, w_v, gsem, wsem):
            pltpu.async_copy(
                feats_hbm.at[idx_v.at[pl.ds(cc * 3 * CQ, 3 * CQ)]],
                rows, gsem)
            pltpu.async_copy(
                w_hbm.at[pl.ds(base_q + cc * CQ, CQ)], w_v, wsem)

        def chunk_wait(cc, rows, w_v, gsem, wsem):
            pltpu.make_async_copy(
                feats_hbm.at[idx_v.at[pl.ds(cc * 3 * CQ, 3 * CQ)]],
                rows, gsem).wait()
            pltpu.make_async_copy(
                w_hbm.at[pl.ds(base_q + cc * CQ, CQ)], w_v, wsem).wait()

        def out_wait(cc, out_v, osem):
            pltpu.make_async_copy(
                out_v, out_hbm.at[pl.ds(base_q + cc * CQ, CQ)], osem).wait()

        def compute_store(cc, rows, w_v, out_v, osem):
            @pl.when(cc >= 2)
            def _():
                out_wait(cc - 2, out_v, osem)

            for q in range(CQ):
                w0 = w_v[q, pl.ds(0, 16)]
                w1 = w_v[q, pl.ds(16, 16)]
                w2 = w_v[q, pl.ds(32, 16)]
                for dc in range(C // _LANES):
                    sl = pl.ds(dc * _LANES, _LANES)
                    acc = rows[3 * q, sl] * w0
                    acc = acc + rows[3 * q + 1, sl] * w1
                    acc = acc + rows[3 * q + 2, sl] * w2
                    out_v[q, sl] = acc
            pltpu.async_copy(
                out_v, out_hbm.at[pl.ds(base_q + cc * CQ, CQ)], osem)

        chunk_start(0, rows_a, w_va, gsem_a, wsem_a)
        chunk_start(1, rows_b, w_vb, gsem_b, wsem_b)

        def pair_body(i, carry):
            cc = 2 * i
            chunk_wait(cc, rows_a, w_va, gsem_a, wsem_a)
            compute_store(cc, rows_a, w_va, out_va, osem_a)

            @pl.when(cc + 2 < nchunk)
            def _():
                chunk_start(cc + 2, rows_a, w_va, gsem_a, wsem_a)

            chunk_wait(cc + 1, rows_b, w_vb, gsem_b, wsem_b)
            compute_store(cc + 1, rows_b, w_vb, out_vb, osem_b)

            @pl.when(cc + 3 < nchunk)
            def _():
                chunk_start(cc + 3, rows_b, w_vb, gsem_b, wsem_b)

            return carry

        lax.fori_loop(0, nchunk // 2, pair_body, 0)
        out_wait(nchunk - 2, out_va, osem_a)
        out_wait(nchunk - 1, out_vb, osem_b)

    return sc_gather


def kernel(higher_feats, lower_points, higher_points):
    B, N, _ = lower_points.shape
    _, M, C = higher_feats.shape
    feats_flat = higher_feats.reshape(B * M, C)
    hpt = jnp.swapaxes(higher_points, 1, 2)
    sc_fn = _make_sc_gather(N, C, 16)
    outs = []
    for b in range(B):
        idx_b, w_b = _tc_stage_batch(lower_points[b], hpt[b], M, b * M)
        outs.append(sc_fn(feats_flat, idx_b.reshape(N * 3),
                          w_b.reshape(N, 48)))
    return jnp.stack(outs, axis=0)
